# baseline (device time: 21237 ns/iter reference)
import jax
import jax.numpy as jnp
from jax import lax
from jax.experimental import pallas as pl
from jax.experimental.pallas import tpu as pltpu

N_DEV = 8
N_TOK = 2048
D_IN = 512
D_OUT = 1024
N_EXP = 32
EXP_PER_DEV = N_EXP // N_DEV
CAP = 51
CAP_PAD = 64
CHUNK = EXP_PER_DEV * CAP_PAD
ROWS_PER_DEV = N_TOK // N_DEV
BLK = 256
N_BLK = N_TOK // BLK
WAIT_CHUNK = 8


def _route_body(route_ref, tok_ref, sched_ref, cnt_ref):
    p = lax.axis_index("i")
    e_row = route_ref[:, :]

    gcol = lax.broadcasted_iota(jnp.int32, (N_EXP, 1), 0)
    onehotT = (e_row == gcol).astype(jnp.float32)
    i0 = lax.broadcasted_iota(jnp.int32, (BLK, BLK), 0)
    i1 = lax.broadcasted_iota(jnp.int32, (BLK, BLK), 1)
    triuS = (i0 < i1).astype(jnp.float32)
    my_slots = (p * CHUNK
                + lax.broadcasted_iota(jnp.int32, (CHUNK, 1), 0)
                ).astype(jnp.float32)
    t0 = (p * ROWS_PER_DEV).astype(jnp.float32)

    off = jnp.zeros((N_EXP, 1), jnp.float32)
    n_recv = jnp.zeros((1, 1), jnp.float32)
    tokp1 = jnp.zeros((CHUNK, 1), jnp.float32)
    for b in range(N_BLK):
        sl = slice(b * BLK, (b + 1) * BLK)
        ohT_b = onehotT[:, sl]
        scan_b = jnp.dot(ohT_b, triuS, preferred_element_type=jnp.float32)
        rank_b = scan_b + off
        off = off + jnp.sum(ohT_b, axis=1, keepdims=True)
        rank_row = jnp.sum(rank_b * ohT_b, axis=0, keepdims=True)
        e_b = e_row[:, sl]
        keep_b = rank_row < CAP
        flat_b = jnp.where(
            keep_b, e_b.astype(jnp.float32) * CAP_PAD + rank_row,
            float(N_EXP * CAP_PAD),
        )
        hits_b = (my_slots == flat_b).astype(jnp.float32)
        tid = (lax.broadcasted_iota(jnp.int32, (1, BLK), 1)
               + b * BLK).astype(jnp.float32)
        tokp1 = tokp1 + jnp.sum(hits_b * (tid + 1.0), axis=1, keepdims=True)
        mine_b = jnp.logical_and(tid >= t0, tid < t0 + ROWS_PER_DEV)
        remote_src = (e_b // EXP_PER_DEV) != p
        n_recv = n_recv + jnp.sum(
            jnp.where(jnp.logical_and(jnp.logical_and(mine_b, keep_b),
                                      remote_src), 1.0, 0.0),
            axis=(0, 1), keepdims=True,
        )

    valid = tokp1 > 0
    tok = jnp.where(valid, tokp1 - 1.0, 0.0)
    tok_ref[:, :] = jnp.where(valid, tok, float(N_TOK)).astype(jnp.int32)
    qf = jnp.floor(tok / ROWS_PER_DEV)
    rf = tok - qf * ROWS_PER_DEV
    q = qf.astype(jnp.int32)
    r = rf.astype(jnp.int32)
    mode = jnp.where(valid, jnp.where(q == p, 1, 2), 0)
    sched_ref[:, :] = mode | (q << 2) | (r << 5)
    n_send = jnp.sum(jnp.where(mode == 2, 1.0, 0.0),
                     axis=(0, 1), keepdims=True)
    cnt_ref[:, :] = jnp.concatenate([n_recv, n_send], axis=0).astype(jnp.int32)


def _dispatch_body(xg_ref, w_ref, sched_ref, cnt_ref, out_ref, y_ref,
                   send_sem, recv_sem):
    p = lax.axis_index("i")

    out_ref[...] = jnp.zeros_like(out_ref)

    for le in range(EXP_PER_DEV):
        y_ref[pl.ds(le * CAP_PAD, CAP_PAD), :] = jnp.dot(
            xg_ref[pl.ds(le * CAP_PAD, CAP_PAD), :].astype(jnp.bfloat16),
            w_ref[le].astype(jnp.bfloat16),
            preferred_element_type=jnp.float32,
        )

    barrier_sem = pltpu.get_barrier_semaphore()
    for k in range(1, N_DEV):
        pl.semaphore_signal(
            barrier_sem, inc=1,
            device_id=(jnp.mod(p + k, N_DEV),),
            device_id_type=pl.DeviceIdType.MESH,
        )
    pl.semaphore_wait(barrier_sem, N_DEV - 1)

    n_recv = cnt_ref[0]
    n_send = cnt_ref[1]

    def send_body(s, carry):
        w = sched_ref[s]
        m = jnp.bitwise_and(w, 3)
        q = jnp.bitwise_and(jnp.right_shift(w, 2), 7)
        r = jnp.right_shift(w, 5)

        @pl.when(m == 2)
        def _():
            rdma = pltpu.make_async_remote_copy(
                src_ref=y_ref.at[pl.ds(s, 1)],
                dst_ref=out_ref.at[pl.ds(r, 1)],
                send_sem=send_sem,
                recv_sem=recv_sem,
                device_id=(q,),
                device_id_type=pl.DeviceIdType.MESH,
            )
            rdma.start()

        @pl.when(m == 1)
        def _():
            out_ref[pl.ds(r, 1), :] = y_ref[pl.ds(s, 1), :]

        return carry

    lax.fori_loop(0, CHUNK, send_body, 0)

    dummy_k = pltpu.make_async_remote_copy(
        src_ref=y_ref.at[pl.ds(0, WAIT_CHUNK)],
        dst_ref=out_ref.at[pl.ds(0, WAIT_CHUNK)],
        send_sem=send_sem, recv_sem=recv_sem,
        device_id=(p,), device_id_type=pl.DeviceIdType.MESH,
    )
    dummy_1 = pltpu.make_async_remote_copy(
        src_ref=y_ref.at[pl.ds(0, 1)],
        dst_ref=out_ref.at[pl.ds(0, 1)],
        send_sem=send_sem, recv_sem=recv_sem,
        device_id=(p,), device_id_type=pl.DeviceIdType.MESH,
    )
    lax.fori_loop(0, n_recv // WAIT_CHUNK,
                  lambda i, c: (dummy_k.wait_recv(), c)[1], 0)
    lax.fori_loop(0, n_recv % WAIT_CHUNK,
                  lambda i, c: (dummy_1.wait_recv(), c)[1], 0)
    lax.fori_loop(0, n_send // WAIT_CHUNK,
                  lambda i, c: (dummy_k.wait_send(), c)[1], 0)
    lax.fori_loop(0, n_send % WAIT_CHUNK,
                  lambda i, c: (dummy_1.wait_send(), c)[1], 0)


def kernel(x, router_W, route_idx, expert_W):
    del router_W

    tok, sched, counts = pl.pallas_call(
        _route_body,
        out_shape=(
            jax.ShapeDtypeStruct((CHUNK, 1), jnp.int32),
            jax.ShapeDtypeStruct((CHUNK, 1), jnp.int32),
            jax.ShapeDtypeStruct((2, 1), jnp.int32),
        ),
        in_specs=[pl.BlockSpec(memory_space=pltpu.VMEM)],
        out_specs=(
            pl.BlockSpec(memory_space=pltpu.VMEM),
            pl.BlockSpec(memory_space=pltpu.VMEM),
            pl.BlockSpec(memory_space=pltpu.VMEM),
        ),
    )(route_idx.reshape(1, N_TOK))

    xg = jnp.take(x, tok.reshape(CHUNK), axis=0, mode="fill", fill_value=0.0)

    return pl.pallas_call(
        _dispatch_body,
        out_shape=jax.ShapeDtypeStruct((ROWS_PER_DEV, D_OUT), jnp.float32),
        in_specs=[
            pl.BlockSpec(memory_space=pltpu.VMEM),
            pl.BlockSpec(memory_space=pltpu.VMEM),
            pl.BlockSpec(memory_space=pltpu.SMEM),
            pl.BlockSpec(memory_space=pltpu.SMEM),
        ],
        out_specs=pl.BlockSpec(memory_space=pltpu.VMEM),
        scratch_shapes=[
            pltpu.VMEM((CHUNK, D_OUT), jnp.float32),
            pltpu.SemaphoreType.DMA,
            pltpu.SemaphoreType.DMA,
        ],
        compiler_params=pltpu.CompilerParams(collective_id=0),
    )(xg, expert_W, sched.reshape(CHUNK), counts.reshape(2))
